# Initial kernel scaffold; baseline (speedup 1.0000x reference)
import jax
import jax.numpy as jnp
from jax.experimental import pallas as pl


def kernel(h, edge_index, e, W_enc, b_enc, W_post0, b_post0, gamma0, beta0, W_post1, b_post1, gamma1, beta1, W_post2, b_post2, gamma2, beta2, W_post3, b_post3):
    raise NotImplementedError("write your pallas kernel here")


# speedup vs baseline: 1.0274x; 1.0274x over previous
import functools

import jax
import jax.numpy as jnp
import numpy as np
from jax import lax
from jax.experimental import pallas as pl
from jax.experimental.pallas import tpu as pltpu

N = 10000
E = 320000
D = 128
HID = 128
NCLS = 64
AVG_DEG = 32.0
DELTA = float(np.log(AVG_DEG + 1.0))


def _encode_body(h_ref, w_ref, b_ref, o_ref):
    o_ref[...] = (
        jnp.dot(h_ref[...], w_ref[...], preferred_element_type=jnp.float32)
        + b_ref[...][None, :]
    )


def _encode(h, W_enc, b_enc):
    return pl.pallas_call(
        _encode_body,
        out_shape=jax.ShapeDtypeStruct((N, HID), jnp.float32),
    )(h, W_enc, b_enc)


def _pna_agg(x, src, dst, deg, degc):
    n = x.shape[0]
    xs = x[src]
    s = jax.ops.segment_sum(xs, dst, num_segments=n)
    mean = s / degc[:, None]
    mx = jax.ops.segment_max(xs, dst, num_segments=n)
    mx = jnp.where(deg[:, None] > 0, mx, 0.0)
    mn = -jax.ops.segment_max(-xs, dst, num_segments=n)
    mn = jnp.where(deg[:, None] > 0, mn, 0.0)
    s2 = jax.ops.segment_sum(xs * xs, dst, num_segments=n)
    var = jnp.maximum(s2 / degc[:, None] - mean * mean, 0.0)
    std = jnp.sqrt(var + 1e-5)
    agg = jnp.concatenate([mean, mx, mn, std], axis=1)
    logd = jnp.log(degc + 1.0)
    amp = (logd / DELTA)[:, None]
    att = (DELTA / logd)[:, None]
    return jnp.concatenate([agg, agg * amp, agg * att], axis=1)


def _bn(x, gamma, beta):
    m = jnp.mean(x, axis=0)
    v = jnp.var(x, axis=0)
    return gamma * (x - m) / jnp.sqrt(v + 1e-5) + beta


def kernel(h, edge_index, e, W_enc, b_enc, W_post0, b_post0, gamma0, beta0, W_post1, b_post1, gamma1, beta1, W_post2, b_post2, gamma2, beta2, W_post3, b_post3):
    src = edge_index[0]
    dst = edge_index[1]

    order = jnp.argsort(dst)
    ds = dst[order]
    ss = src[order]
    row_ptr = jnp.searchsorted(ds, jnp.arange(N + 1, dtype=jnp.int32)).astype(jnp.int32)
    deg = (row_ptr[1:] - row_ptr[:-1]).astype(jnp.float32)
    degc = jnp.maximum(deg, 1.0)

    hx = _encode(h, W_enc, b_enc)

    Ws = [W_post0, W_post1, W_post2]
    bs = [b_post0, b_post1, b_post2]
    gs = [gamma0, gamma1, gamma2]
    bt = [beta0, beta1, beta2]
    for i in range(3):
        a = _pna_agg(hx, ss, ds, deg, degc)
        out = a @ Ws[i] + bs[i]
        out = _bn(out, gs[i], bt[i])
        out = jax.nn.relu(out)
        hx = out + hx
    a = _pna_agg(hx, ss, ds, deg, degc)
    out = a @ W_post3 + b_post3
    return out


# speedup vs baseline: 4.7914x; 4.6634x over previous
import functools

import jax
import jax.numpy as jnp
import numpy as np
from jax import lax
from jax.experimental import pallas as pl
from jax.experimental.pallas import tpu as pltpu
from jax.experimental.pallas import tpu_sc as plsc

N = 10000
E = 320000
D = 128
HID = 128
NCLS = 64
AVG_DEG = 32.0
DELTA = float(np.log(AVG_DEG + 1.0))

NW = 32
EPW = E // NW
CH = 128
NFULL = EPW // CH
TAIL = EPW - NFULL * CH
NSTG = 8

B = 200
NB = N // B


def _sc_agg(hx, ss, ds):
    mesh = plsc.VectorSubcoreMesh(core_axis_name="c", subcore_axis_name="s")
    info = plsc.get_sparse_core_info()
    nc = info.num_cores

    @functools.partial(
        pl.kernel,
        mesh=mesh,
        out_type=(
            jax.ShapeDtypeStruct((N, 512), jnp.float32),
            jax.ShapeDtypeStruct((2 * NW, 528), jnp.float32),
        ),
        scratch_types=[
            pltpu.VMEM((CH,), jnp.int32),
            pltpu.VMEM((CH + 16,), jnp.int32),
            pltpu.VMEM((CH, D), jnp.float32),
            pltpu.VMEM((NSTG, 512), jnp.float32),
            pltpu.VMEM((1, 528), jnp.float32),
            pltpu.SemaphoreType.DMA,
            pltpu.SemaphoreType.DMA,
        ],
    )
    def agg(hx_hbm, ss_hbm, ds_hbm, stats_hbm, part_hbm, ssv, dsv, rows,
            stg, pbuf, gsem, fsem):
        wid = lax.axis_index("s") * nc + lax.axis_index("c")
        e0 = wid * EPW
        zero = jnp.zeros((16,), jnp.float32)
        neg = jnp.full((16,), -jnp.inf, jnp.float32)
        pos = jnp.full((16,), jnp.inf, jnp.float32)
        neutral = (
            tuple(zero for _ in range(8)),
            tuple(neg for _ in range(8)),
            tuple(pos for _ in range(8)),
            tuple(zero for _ in range(8)),
        )

        def write_row(row_ref, accs):
            sm, mx, mn, sq = accs
            for k in range(8):
                row_ref[pl.ds(k * 16, 16)] = sm[k]
                row_ref[pl.ds(128 + k * 16, 16)] = mx[k]
                row_ref[pl.ds(256 + k * 16, 16)] = mn[k]
                row_ref[pl.ds(384 + k * 16, 16)] = sq[k]

        def part_flush(row_idx, accs, dval):
            prow = pbuf.at[0]
            write_row(prow, accs)
            prow[pl.ds(512, 16)] = jnp.full((16,), 1.0, jnp.float32) * dval
            pltpu.sync_copy(pbuf.at[0], part_hbm.at[row_idx])

        def flush_reset(carry, d_j):
            accs, cur_d, run_idx, slot, pending = carry

            def first_run(sp):
                s_, p_ = sp
                part_flush(2 * wid, accs, cur_d.astype(jnp.float32))
                return s_, p_

            def interior(sp):
                s_, p_ = sp

                def drain(p2):
                    pltpu.make_async_copy(
                        stats_hbm.at[0], stg.at[0], fsem).wait()
                    return p2 - 1

                p_ = lax.cond(p_ >= NSTG, drain, lambda p2: p2, p_)
                write_row(stg.at[s_], accs)
                pltpu.async_copy(stg.at[s_], stats_hbm.at[cur_d], fsem)
                return lax.rem(s_ + 1, NSTG), p_ + 1

            def skip(sp):
                return sp

            slot, pending = lax.cond(
                run_idx < 0, skip,
                lambda sp: lax.cond(run_idx == 0, first_run, interior, sp),
                (slot, pending))
            return (neutral, d_j, run_idx + 1, slot, pending)

        def edge_body(j, carry):
            d_j = dsv[pl.ds(j, 16)][0]
            carry = lax.cond(
                d_j != carry[1],
                lambda c: flush_reset(c, d_j),
                lambda c: c,
                carry)
            accs, cur_d, run_idx, slot, pending = carry
            sm, mx, mn, sq = accs
            rrow = rows.at[j]
            sm2, mx2, mn2, sq2 = [], [], [], []
            for k in range(8):
                r = rrow[pl.ds(k * 16, 16)]
                sm2.append(sm[k] + r)
                sq2.append(sq[k] + r * r)
                mx2.append(jnp.maximum(mx[k], r))
                mn2.append(jnp.minimum(mn[k], r))
            accs = (tuple(sm2), tuple(mx2), tuple(mn2), tuple(sq2))
            return (accs, cur_d, run_idx, slot, pending)

        def process(base, size, carry):
            pltpu.sync_copy(ss_hbm.at[pl.ds(base, size)],
                            ssv.at[pl.ds(0, size)])
            pltpu.sync_copy(ds_hbm.at[pl.ds(base, size)],
                            dsv.at[pl.ds(0, size)])
            pltpu.async_copy(hx_hbm.at[ssv.at[pl.ds(0, size)]],
                             rows.at[pl.ds(0, size)], gsem).wait()
            return lax.fori_loop(0, size, edge_body, carry)

        carry0 = (neutral, jnp.int32(-1), jnp.int32(-1), jnp.int32(0),
                  jnp.int32(0))

        def chunk_body(c, carry):
            return process(e0 + c * CH, CH, carry)

        carry = lax.fori_loop(0, NFULL, chunk_body, carry0)
        if TAIL:
            carry = process(e0 + NFULL * CH, TAIL, carry)
        accs, cur_d, run_idx, slot, pending = carry

        def one_run(_):
            part_flush(2 * wid, accs, cur_d.astype(jnp.float32))
            part_flush(2 * wid + 1, neutral, jnp.float32(N))
            return 0

        def multi_run(_):
            part_flush(2 * wid + 1, accs, cur_d.astype(jnp.float32))
            return 0

        lax.cond(run_idx == 0, one_run, multi_run, 0)

        def drain_body(i, p):
            pltpu.make_async_copy(stats_hbm.at[0], stg.at[0], fsem).wait()
            return p

        lax.fori_loop(0, pending, drain_body, pending)

    return agg(hx, ss, ds)


def _merge_body(part_ref, o_ref):
    dval = part_ref[:, 512:513]
    sums = part_ref[:, 0:128]
    mxs = part_ref[:, 128:256]
    mns = part_ref[:, 256:384]
    sqs = part_ref[:, 384:512]
    valid = dval < float(N)
    for i in range(2 * NW):
        m = (dval == dval[i, 0]) & valid
        s = jnp.sum(jnp.where(m, sums, 0.0), axis=0, keepdims=True)
        mx = jnp.max(jnp.where(m, mxs, -jnp.inf), axis=0, keepdims=True)
        mn = jnp.min(jnp.where(m, mns, jnp.inf), axis=0, keepdims=True)
        sq = jnp.sum(jnp.where(m, sqs, 0.0), axis=0, keepdims=True)
        o_ref[pl.ds(i, 1), 0:128] = s
        o_ref[pl.ds(i, 1), 128:256] = mx
        o_ref[pl.ds(i, 1), 256:384] = mn
        o_ref[pl.ds(i, 1), 384:512] = sq


def _merge(partials):
    return pl.pallas_call(
        _merge_body,
        out_shape=jax.ShapeDtypeStruct((2 * NW, 512), jnp.float32),
    )(partials)


def _patch_and_agg(stats_ref, merged_ref, pdm_ref, dvec_ref, step):
    for i in range(2 * NW):
        d = pdm_ref[i]
        loc = d - step * B

        @pl.when((loc >= 0) & (loc < B))
        def _():
            stats_ref[pl.ds(loc, 1), :] = merged_ref[pl.ds(i, 1), :]

    inv = dvec_ref[:, 0:1]
    amp = dvec_ref[:, 1:2]
    att = dvec_ref[:, 2:3]
    zm = dvec_ref[:, 3:4] > 0.0
    s = stats_ref[:, 0:128]
    mx = stats_ref[:, 128:256]
    mn = stats_ref[:, 256:384]
    sq = stats_ref[:, 384:512]
    mean = jnp.where(zm, s * inv, 0.0)
    var = jnp.where(zm, jnp.maximum(sq * inv - mean * mean, 0.0), 0.0)
    std = jnp.sqrt(var + 1e-5)
    agg = jnp.concatenate(
        [mean, jnp.where(zm, mx, 0.0), jnp.where(zm, mn, 0.0), std], axis=1)
    return agg, amp, att


def _dense_body(stats_ref, merged_ref, pdm_ref, dvec_ref, w_ref, b_ref,
                opre_ref, osum_ref, acc_ref):
    step = pl.program_id(0)
    agg, amp, att = _patch_and_agg(stats_ref, merged_ref, pdm_ref, dvec_ref,
                                   step)
    o = (jnp.dot(agg, w_ref[0], preferred_element_type=jnp.float32)
         + amp * jnp.dot(agg, w_ref[1], preferred_element_type=jnp.float32)
         + att * jnp.dot(agg, w_ref[2], preferred_element_type=jnp.float32)
         + b_ref[...][None, :])
    opre_ref[...] = o

    @pl.when(step == 0)
    def _():
        acc_ref[...] = jnp.zeros_like(acc_ref)

    acc_ref[0:1, :] += jnp.sum(o, axis=0, keepdims=True)
    acc_ref[1:2, :] += jnp.sum(o * o, axis=0, keepdims=True)
    osum_ref[...] = acc_ref[...]


def _dense(stats, merged, pdm, dvec, w3, b):
    return pl.pallas_call(
        _dense_body,
        grid=(NB,),
        in_specs=[
            pl.BlockSpec((B, 512), lambda i: (i, 0)),
            pl.BlockSpec((2 * NW, 512), lambda i: (0, 0)),
            pl.BlockSpec(memory_space=pltpu.SMEM),
            pl.BlockSpec((B, 4), lambda i: (i, 0)),
            pl.BlockSpec((3, 512, HID), lambda i: (0, 0, 0)),
            pl.BlockSpec((HID,), lambda i: (0,)),
        ],
        out_specs=[
            pl.BlockSpec((B, HID), lambda i: (i, 0)),
            pl.BlockSpec((8, HID), lambda i: (0, 0)),
        ],
        out_shape=[
            jax.ShapeDtypeStruct((N, HID), jnp.float32),
            jax.ShapeDtypeStruct((8, HID), jnp.float32),
        ],
        scratch_shapes=[pltpu.VMEM((8, HID), jnp.float32)],
    )(stats, merged, pdm, dvec, w3, b)


def _bn_body(opre_ref, osum_ref, hx_ref, g_ref, bt_ref, o_ref):
    m = osum_ref[0:1, :] / N
    v = osum_ref[1:2, :] / N - m * m
    o = (g_ref[...][None, :] * (opre_ref[...] - m)
         / jnp.sqrt(v + 1e-5) + bt_ref[...][None, :])
    o_ref[...] = jnp.maximum(o, 0.0) + hx_ref[...]


def _bn_apply(opre, osum, hx, g, bt):
    return pl.pallas_call(
        _bn_body,
        grid=(NB,),
        in_specs=[
            pl.BlockSpec((B, HID), lambda i: (i, 0)),
            pl.BlockSpec((8, HID), lambda i: (0, 0)),
            pl.BlockSpec((B, HID), lambda i: (i, 0)),
            pl.BlockSpec((HID,), lambda i: (0,)),
            pl.BlockSpec((HID,), lambda i: (0,)),
        ],
        out_specs=pl.BlockSpec((B, HID), lambda i: (i, 0)),
        out_shape=jax.ShapeDtypeStruct((N, HID), jnp.float32),
    )(opre, osum, hx, g, bt)


def _final_body(stats_ref, merged_ref, pdm_ref, dvec_ref, w_ref, b_ref,
                o_ref):
    step = pl.program_id(0)
    agg, amp, att = _patch_and_agg(stats_ref, merged_ref, pdm_ref, dvec_ref,
                                   step)
    o_ref[...] = (
        jnp.dot(agg, w_ref[0], preferred_element_type=jnp.float32)
        + amp * jnp.dot(agg, w_ref[1], preferred_element_type=jnp.float32)
        + att * jnp.dot(agg, w_ref[2], preferred_element_type=jnp.float32)
        + b_ref[...][None, :])


def _final(stats, merged, pdm, dvec, w3, b):
    return pl.pallas_call(
        _final_body,
        grid=(NB,),
        in_specs=[
            pl.BlockSpec((B, 512), lambda i: (i, 0)),
            pl.BlockSpec((2 * NW, 512), lambda i: (0, 0)),
            pl.BlockSpec(memory_space=pltpu.SMEM),
            pl.BlockSpec((B, 4), lambda i: (i, 0)),
            pl.BlockSpec((3, 512, NCLS), lambda i: (0, 0, 0)),
            pl.BlockSpec((NCLS,), lambda i: (0,)),
        ],
        out_specs=pl.BlockSpec((B, NCLS), lambda i: (i, 0)),
        out_shape=jax.ShapeDtypeStruct((N, NCLS), jnp.float32),
    )(stats, merged, pdm, dvec, w3, b)


def _encode_body(h_ref, w_ref, b_ref, o_ref):
    o_ref[...] = (
        jnp.dot(h_ref[...], w_ref[...], preferred_element_type=jnp.float32)
        + b_ref[...][None, :])


def _encode(h, W_enc, b_enc):
    return pl.pallas_call(
        _encode_body,
        grid=(NB,),
        in_specs=[
            pl.BlockSpec((B, D), lambda i: (i, 0)),
            pl.BlockSpec((D, HID), lambda i: (0, 0)),
            pl.BlockSpec((HID,), lambda i: (0,)),
        ],
        out_specs=pl.BlockSpec((B, HID), lambda i: (i, 0)),
        out_shape=jax.ShapeDtypeStruct((N, HID), jnp.float32),
    )(h, W_enc, b_enc)


def kernel(h, edge_index, e, W_enc, b_enc, W_post0, b_post0, gamma0, beta0,
           W_post1, b_post1, gamma1, beta1, W_post2, b_post2, gamma2, beta2,
           W_post3, b_post3):
    src = edge_index[0]
    dst = edge_index[1]

    order = jnp.argsort(dst)
    ds = dst[order]
    ss = src[order]
    row_ptr = jnp.searchsorted(
        ds, jnp.arange(N + 1, dtype=jnp.int32)).astype(jnp.int32)
    deg = (row_ptr[1:] - row_ptr[:-1]).astype(jnp.float32)
    degc = jnp.maximum(deg, 1.0)
    logd = jnp.log(degc + 1.0)
    dvec = jnp.stack(
        [1.0 / degc, logd / DELTA, DELTA / logd,
         (deg > 0).astype(jnp.float32)], axis=1)

    hx = _encode(h, W_enc, b_enc)

    Ws = [W_post0, W_post1, W_post2]
    bs = [b_post0, b_post1, b_post2]
    gs = [gamma0, gamma1, gamma2]
    bt = [beta0, beta1, beta2]
    for i in range(3):
        stats, partials = _sc_agg(hx, ss, ds)
        merged = _merge(partials)
        pdm = partials[:, 512].astype(jnp.int32)
        w3 = Ws[i].reshape(3, 512, HID)
        opre, osum = _dense(stats, merged, pdm, dvec, w3, bs[i])
        hx = _bn_apply(opre, osum, hx, gs[i], bt[i])
    stats, partials = _sc_agg(hx, ss, ds)
    merged = _merge(partials)
    pdm = partials[:, 512].astype(jnp.int32)
    w3 = W_post3.reshape(3, 512, NCLS)
    return _final(stats, merged, pdm, dvec, w3, b_post3)
